# baseline (speedup 1.0000x reference)
import functools

import jax
import jax.numpy as jnp
from jax import lax
from jax.experimental import pallas as pl
from jax.experimental.pallas import tpu as pltpu
from jax.experimental.pallas import tpu_sc as plsc

N_NUM = 10
N_CAT = 26
N_TOT = N_CAT + N_NUM
B = 4096
D = 128
VOCAB = 1000

NC = 2
NS = 16
NW = NC * NS
BPW = B // NW

N_BUF = 7
LOOKAHEAD = 5

_mesh = plsc.VectorSubcoreMesh(
    core_axis_name="c", subcore_axis_name="s", num_cores=NC, num_subcores=NS
)


@functools.partial(
    pl.kernel,
    out_type=jax.ShapeDtypeStruct((N_TOT, B, D), jnp.float32),
    mesh=_mesh,
    scratch_types=(
        [pltpu.VMEM((N_CAT, BPW), jnp.int32)]
        + [pltpu.VMEM((BPW, D), jnp.float32) for _ in range(N_BUF)]
        + [pltpu.SemaphoreType.DMA for _ in range(2 * N_BUF)]
    ),
)
def _gather_kernel(tables, idx, out, idx_v, *rest):
    gbufs = rest[:N_BUF]
    gsems = rest[N_BUF:2 * N_BUF]
    wsems = rest[2 * N_BUF:]

    wid = lax.axis_index("s") * NC + lax.axis_index("c")
    b0 = wid * BPW

    pltpu.sync_copy(idx.at[:, pl.ds(b0, BPW)], idx_v)

    def gather(f):
        return pltpu.async_copy(
            tables.at[f].at[idx_v.at[f]], gbufs[f % N_BUF], gsems[f % N_BUF]
        )

    gc = [None] * N_CAT
    wc = [None] * N_CAT
    waited = set()
    for f in range(LOOKAHEAD):
        gc[f] = gather(f)
    for f in range(N_CAT):
        nf = f + LOOKAHEAD
        if nf < N_CAT:
            if nf - N_BUF >= 0:
                wc[nf - N_BUF].wait()
                waited.add(nf - N_BUF)
            gc[nf] = gather(nf)
        gc[f].wait()
        wc[f] = pltpu.async_copy(
            gbufs[f % N_BUF], out.at[f, pl.ds(b0, BPW)], wsems[f % N_BUF]
        )
    for f in range(N_CAT):
        if f not in waited:
            wc[f].wait()


BH = B // 4


def _num_body(x_ref, w_ref, cat_ref, o_ref):
    del cat_ref
    o_ref[0] = x_ref[...][:, None] * w_ref[...][None, :]


_num_call = pl.pallas_call(
    _num_body,
    out_shape=jax.ShapeDtypeStruct((N_TOT, B, D), jnp.float32),
    grid=(N_NUM, B // BH),
    in_specs=[
        pl.BlockSpec((BH,), lambda j, h: (j * (B // BH) + h,)),
        pl.BlockSpec((D,), lambda j, h: (j,)),
        pl.BlockSpec((1, 8, D), lambda j, h: (0, 0, 0)),
    ],
    out_specs=pl.BlockSpec((1, BH, D), lambda j, h: (N_CAT + j, h, 0)),
    input_output_aliases={2: 0},
)


def kernel(num_features, cat_features, cat_tables, num_weights):
    idx = cat_features.astype(jnp.int32)
    xs = num_features.reshape(N_NUM * B)
    ws = num_weights.reshape(N_NUM * D)
    cat = _gather_kernel(cat_tables, idx)
    out = _num_call(xs, ws, cat)
    return jnp.transpose(out, (1, 0, 2))


# speedup vs baseline: 1.1706x; 1.1706x over previous
import functools

import jax
import jax.numpy as jnp
from jax import lax
from jax.experimental import pallas as pl
from jax.experimental.pallas import tpu as pltpu
from jax.experimental.pallas import tpu_sc as plsc

N_NUM = 10
N_CAT = 26
N_TOT = N_CAT + N_NUM
B = 4096
D = 128
VOCAB = 1000

NC = 2
NS = 16
NW = NC * NS
BPW = B // NW

N_BUF = 7
LOOKAHEAD = 5

_mesh = plsc.VectorSubcoreMesh(
    core_axis_name="c", subcore_axis_name="s", num_cores=NC, num_subcores=NS
)


@functools.partial(
    pl.kernel,
    out_type=jax.ShapeDtypeStruct((N_TOT, B, D), jnp.float32),
    mesh=_mesh,
    scratch_types=(
        [pltpu.VMEM((N_CAT, BPW), jnp.int32)]
        + [pltpu.VMEM((BPW, D), jnp.float32) for _ in range(N_BUF)]
        + [pltpu.SemaphoreType.DMA for _ in range(2 * N_BUF)]
    ),
)
def _gather_kernel(tables, idx, out, idx_v, *rest):
    gbufs = rest[:N_BUF]
    gsems = rest[N_BUF:2 * N_BUF]
    wsems = rest[2 * N_BUF:]

    wid = lax.axis_index("c") * NS + lax.axis_index("s")
    b0 = wid * BPW

    pltpu.sync_copy(idx.at[:, pl.ds(b0, BPW)], idx_v)

    def gather(f):
        return pltpu.async_copy(
            tables.at[f].at[idx_v.at[f]], gbufs[f % N_BUF], gsems[f % N_BUF]
        )

    gc = [None] * N_CAT
    wc = [None] * N_CAT
    waited = set()
    for f in range(LOOKAHEAD):
        gc[f] = gather(f)
    for f in range(N_CAT):
        nf = f + LOOKAHEAD
        if nf < N_CAT:
            if nf - N_BUF >= 0:
                wc[nf - N_BUF].wait()
                waited.add(nf - N_BUF)
            gc[nf] = gather(nf)
        gc[f].wait()
        wc[f] = pltpu.async_copy(
            gbufs[f % N_BUF], out.at[f, pl.ds(b0, BPW)], wsems[f % N_BUF]
        )
    for f in range(N_CAT):
        if f not in waited:
            wc[f].wait()


def _num_body(x_ref, w_ref, cat_ref, o_ref):
    del cat_ref
    o_ref[0] = x_ref[...][:, None] * w_ref[...][None, :]


_num_call = pl.pallas_call(
    _num_body,
    out_shape=jax.ShapeDtypeStruct((N_TOT, B, D), jnp.float32),
    grid=(N_NUM,),
    in_specs=[
        pl.BlockSpec((B,), lambda j: (j,)),
        pl.BlockSpec((D,), lambda j: (j,)),
        pl.BlockSpec((1, 8, D), lambda j: (0, 0, 0)),
    ],
    out_specs=pl.BlockSpec((1, B, D), lambda j: (N_CAT + j, 0, 0)),
    input_output_aliases={2: 0},
)


def kernel(num_features, cat_features, cat_tables, num_weights):
    idx = cat_features.astype(jnp.int32)
    xs = num_features.reshape(N_NUM * B)
    ws = num_weights.reshape(N_NUM * D)
    cat = _gather_kernel(cat_tables, idx)
    out = _num_call(xs, ws, cat)
    return jnp.transpose(out, (1, 0, 2))


# speedup vs baseline: 1.1789x; 1.0071x over previous
import functools

import jax
import jax.numpy as jnp
from jax import lax
from jax.experimental import pallas as pl
from jax.experimental.pallas import tpu as pltpu
from jax.experimental.pallas import tpu_sc as plsc

N_NUM = 10
N_CAT = 26
N_TOT = N_CAT + N_NUM
B = 4096
D = 128
VOCAB = 1000

NC = 2
NS = 16
NW = NC * NS
BPW = B // NW

N_BUF = 7
LOOKAHEAD = 5

_mesh = plsc.VectorSubcoreMesh(
    core_axis_name="c", subcore_axis_name="s", num_cores=NC, num_subcores=NS
)


@functools.partial(
    pl.kernel,
    out_type=jax.ShapeDtypeStruct((N_TOT, B, D), jnp.float32),
    mesh=_mesh,
    scratch_types=(
        [pltpu.VMEM((N_CAT, BPW), jnp.int32)]
        + [pltpu.VMEM((BPW, D), jnp.float32) for _ in range(N_BUF)]
        + [pltpu.SemaphoreType.DMA for _ in range(2 * N_BUF)]
    ),
)
def _gather_kernel(tables, idx, out, idx_v, *rest):
    gbufs = rest[:N_BUF]
    gsems = rest[N_BUF:2 * N_BUF]
    wsems = rest[2 * N_BUF:]

    wid = lax.axis_index("s") * NC + lax.axis_index("c")
    b0 = wid * BPW

    pltpu.sync_copy(idx.at[:, pl.ds(b0, BPW)], idx_v)

    def gather(f):
        return pltpu.async_copy(
            tables.at[f].at[idx_v.at[f]], gbufs[f % N_BUF], gsems[f % N_BUF]
        )

    gc = [None] * N_CAT
    wc = [None] * N_CAT
    waited = set()
    for f in range(LOOKAHEAD):
        gc[f] = gather(f)
    for f in range(N_CAT):
        nf = f + LOOKAHEAD
        if nf < N_CAT:
            if nf - N_BUF >= 0:
                wc[nf - N_BUF].wait()
                waited.add(nf - N_BUF)
            gc[nf] = gather(nf)
        gc[f].wait()
        wc[f] = pltpu.async_copy(
            gbufs[f % N_BUF], out.at[f, pl.ds(b0, BPW)], wsems[f % N_BUF]
        )
    for f in range(N_CAT):
        if f not in waited:
            wc[f].wait()


def _num_body(x_ref, w_ref, cat_ref, o_ref):
    del cat_ref
    o_ref[0] = x_ref[...][:, None] * w_ref[...][None, :]


_num_call = pl.pallas_call(
    _num_body,
    out_shape=jax.ShapeDtypeStruct((N_TOT, B, D), jnp.float32),
    grid=(N_NUM,),
    in_specs=[
        pl.BlockSpec((B,), lambda j: (j,)),
        pl.BlockSpec((D,), lambda j: (j,)),
        pl.BlockSpec((1, 8, D), lambda j: (0, 0, 0)),
    ],
    out_specs=pl.BlockSpec((1, B, D), lambda j: (N_CAT + j, 0, 0)),
    input_output_aliases={2: 0},
)


def kernel(num_features, cat_features, cat_tables, num_weights):
    idx = cat_features.astype(jnp.int32)
    xs = num_features.reshape(N_NUM * B)
    ws = num_weights.reshape(N_NUM * D)
    cat = _gather_kernel(cat_tables, idx)
    out = _num_call(xs, ws, cat)
    return jnp.transpose(out, (1, 0, 2))
